# baseline (speedup 1.0000x reference)
import functools

import jax
import jax.numpy as jnp
import numpy as np
from jax import lax
from jax.experimental import pallas as pl
from jax.experimental.pallas import tpu as pltpu
from jax.experimental.pallas import tpu_sc as plsc

NC = 2
NS = 16
NW = NC * NS
LANES = 16


def _a_perm(D):
    return np.concatenate(
        [np.concatenate([g * 32 + np.arange(0, 32, 2),
                         g * 32 + np.arange(1, 32, 2)])
         for g in range(D // 32)])


def _rne_bf16_bits(u):
    return (u + jnp.uint32(0x7FFF) + ((u >> 16) & jnp.uint32(1))) & jnp.uint32(
        0xFFFF0000)


def _pack_bf16(v, e_even, e_odd):
    pe = jnp.dot(v, e_even, preferred_element_type=jnp.float32)
    po = jnp.dot(v, e_odd, preferred_element_type=jnp.float32)
    ue = jax.lax.bitcast_convert_type(pe, jnp.uint32)
    uo = jax.lax.bitcast_convert_type(po, jnp.uint32)
    w = (_rne_bf16_bits(ue) >> 16) | _rne_bf16_bits(uo)
    return jax.lax.bitcast_convert_type(w, jnp.float32)


def _ea_proj_body(attr_ref, w_ref, b_ref, ee_ref, eo_ref, o0, o1, o2):
    for j, o in enumerate((o0, o1, o2)):
        ea = (jnp.dot(attr_ref[...], w_ref[..., j, :],
                      preferred_element_type=jnp.float32) + b_ref[..., j, :])
        o[...] = _pack_bf16(ea, ee_ref[...], eo_ref[...])


def _ea_proj3(edge_attr, ws, bs, e_even, e_odd, blk=2000):
    E, ED = edge_attr.shape
    D = ws[0].shape[1]
    half = pl.BlockSpec((blk, D // 2), lambda i: (i, 0))
    outs = pl.pallas_call(
        _ea_proj_body,
        grid=(E // blk,),
        in_specs=[
            pl.BlockSpec((blk, ED), lambda i: (i, 0)),
            pl.BlockSpec((ED, 3, D), lambda i: (0, 0, 0)),
            pl.BlockSpec((1, 3, D), lambda i: (0, 0, 0)),
            pl.BlockSpec((D, D // 2), lambda i: (0, 0)),
            pl.BlockSpec((D, D // 2), lambda i: (0, 0)),
        ],
        out_specs=[half, half, half],
        out_shape=[jax.ShapeDtypeStruct((E, D // 2), jnp.float32)] * 3,
    )(edge_attr, jnp.stack(ws, axis=1), jnp.stack(bs, axis=0).reshape(1, 3, D),
      e_even, e_odd)
    return outs


def _ea_proj1_body(attr_ref, w_ref, b_ref, ee_ref, eo_ref, out_ref):
    ea = (jnp.dot(attr_ref[...], w_ref[...], preferred_element_type=jnp.float32)
          + b_ref[...])
    out_ref[...] = _pack_bf16(ea, ee_ref[...], eo_ref[...])


def _ea_proj1(edge_attr, w, b, e_even, e_odd, blk=2000):
    E, ED = edge_attr.shape
    D = w.shape[1]
    return pl.pallas_call(
        _ea_proj1_body,
        grid=(E // blk,),
        in_specs=[
            pl.BlockSpec((blk, ED), lambda i: (i, 0)),
            pl.BlockSpec((ED, D), lambda i: (0, 0)),
            pl.BlockSpec((1, D), lambda i: (0, 0)),
            pl.BlockSpec((D, D // 2), lambda i: (0, 0)),
            pl.BlockSpec((D, D // 2), lambda i: (0, 0)),
        ],
        out_specs=pl.BlockSpec((blk, D // 2), lambda i: (i, 0)),
        out_shape=jax.ShapeDtypeStruct((E, D // 2), jnp.float32),
    )(edge_attr, w, b.reshape(1, D), e_even, e_odd)


def _prep_body(x_ref, ea_ref, xa_ref):
    xa_ref[...] = jnp.dot(x_ref[...], ea_ref[...],
                          preferred_element_type=jnp.float32)


def _prep(x, e_a, blk=1000):
    N, D = x.shape
    mat = pl.BlockSpec((D, D), lambda i: (0, 0))
    rows = pl.BlockSpec((blk, D), lambda i: (i, 0))
    return pl.pallas_call(
        _prep_body,
        grid=(N // blk,),
        in_specs=[rows, mat],
        out_specs=rows,
        out_shape=jax.ShapeDtypeStruct((N, D), jnp.float32),
    )(x, e_a)


def _mlp_mid_body(x_ref, a0_ref, a1_ref, w0a, b0, w1, b1, w2a, b2a, ha_ref):
    h = x_ref[...] + a0_ref[...] + a1_ref[...]
    h = jnp.maximum(
        jnp.dot(h, w0a[...], preferred_element_type=jnp.float32) + b0[...], 0.0)
    h = jnp.maximum(
        jnp.dot(h, w1[...], preferred_element_type=jnp.float32) + b1[...], 0.0)
    h = jnp.dot(h, w2a[...], preferred_element_type=jnp.float32) + b2a[...]
    ha_ref[...] = jnp.maximum(h, 0.0)


def _mlp_mid(xa, a0, a1, p, A, blk=1000):
    N, D = xa.shape
    mat = pl.BlockSpec((D, D), lambda i: (0, 0))
    vec = pl.BlockSpec((1, D), lambda i: (0, 0))
    rows = pl.BlockSpec((blk, D), lambda i: (i, 0))
    return pl.pallas_call(
        _mlp_mid_body,
        grid=(N // blk,),
        in_specs=[rows, rows, rows, mat, vec, mat, vec, mat, vec],
        out_specs=rows,
        out_shape=jax.ShapeDtypeStruct((N, D), jnp.float32),
    )(xa, a0, a1,
      p['W0'][A, :], p['b0'].reshape(1, D),
      p['W1'], p['b1'].reshape(1, D),
      p['W2'][:, A], p['b2'][A].reshape(1, D))


def _mlp_last_body(x_ref, a0_ref, a1_ref, w0a, b0, w1, b1, w2, b2, out_ref):
    h = x_ref[...] + a0_ref[...] + a1_ref[...]
    h = jnp.maximum(
        jnp.dot(h, w0a[...], preferred_element_type=jnp.float32) + b0[...], 0.0)
    h = jnp.maximum(
        jnp.dot(h, w1[...], preferred_element_type=jnp.float32) + b1[...], 0.0)
    out_ref[...] = (
        jnp.dot(h, w2[...], preferred_element_type=jnp.float32) + b2[...])


def _mlp_last(xa, a0, a1, p, A, blk=1000):
    N, D = xa.shape
    mat = pl.BlockSpec((D, D), lambda i: (0, 0))
    vec = pl.BlockSpec((1, D), lambda i: (0, 0))
    rows = pl.BlockSpec((blk, D), lambda i: (i, 0))
    return pl.pallas_call(
        _mlp_last_body,
        grid=(N // blk,),
        in_specs=[rows, rows, rows, mat, vec, mat, vec, mat, vec],
        out_specs=rows,
        out_shape=jax.ShapeDtypeStruct((N, D), jnp.float32),
    )(xa, a0, a1,
      p['W0'][A, :], p['b0'].reshape(1, D),
      p['W1'], p['b1'].reshape(1, D),
      p['W2'], p['b2'].reshape(1, D))


@functools.lru_cache(maxsize=None)
def _sc_gather_scatter_fn(N, D, C, NCH):
    NB = 4
    NBI = 8
    PF = NB - 1
    ZC = C
    NZ = N // ZC
    ZT = (NZ + NS - 1) // NS
    DV = D // LANES
    T = (NCH + NBI - 1) // NBI

    mesh = plsc.VectorSubcoreMesh(core_axis_name="c", subcore_axis_name="s")

    @functools.partial(
        pl.kernel,
        out_type=[jax.ShapeDtypeStruct((N, D), jnp.float32),
                  jax.ShapeDtypeStruct((N, D), jnp.float32)],
        mesh=mesh,
        scratch_types=[
            pltpu.VMEM((NBI, 2, C), jnp.int32),
            pltpu.VMEM((NB, C, D // 2), jnp.float32),
            pltpu.VMEM((NB, C, D), jnp.float32),
            pltpu.VMEM_SHARED((N, D), jnp.float32),
            [pltpu.SemaphoreType.DMA] * NBI,
            [pltpu.SemaphoreType.DMA] * NB,
            [pltpu.SemaphoreType.DMA] * NB,
        ],
    )
    def k(x_hbm, idx_hbm, ea_hbm, out0, out1,
          idx_v, ea_v, g_v, agg_sh, sem_idx, sem_in, sem_s):
        c = lax.axis_index("c")
        s = lax.axis_index("s")
        wid = s * NC + c
        base = wid * NCH

        def issue_idx(g, bi):
            pltpu.async_copy(idx_hbm.at[wid, g], idx_v.at[bi], sem_idx[bi])

        def wait_idx(bi):
            pltpu.make_async_copy(idx_hbm.at[0, 0], idx_v.at[bi],
                                  sem_idx[bi]).wait()

        def issue_loads(g, b, bi):
            pltpu.async_copy(ea_hbm.at[pl.ds((base + g) * C, C)], ea_v.at[b],
                             sem_in[b])
            pltpu.async_copy(x_hbm.at[idx_v.at[bi, 0]], g_v.at[b], sem_in[b])

        def wait_loads(b):
            pltpu.make_async_copy(ea_hbm.at[pl.ds(0, C)], ea_v.at[b],
                                  sem_in[b]).wait()
            pltpu.make_async_copy(out0.at[pl.ds(0, C)], g_v.at[b],
                                  sem_in[b]).wait()

        def issue_scatter(b, bi):
            pltpu.async_copy(g_v.at[b], agg_sh.at[idx_v.at[bi, 1]], sem_s[b],
                             add=True)

        def wait_scatter(b):
            pltpu.make_async_copy(out0.at[pl.ds(0, C)], g_v.at[b],
                                  sem_s[b]).wait()

        zero = jnp.zeros((LANES,), jnp.float32)

        def zrow(r, carry):
            for j in range(DV):
                g_v[0, r, pl.ds(j * LANES, LANES)] = zero
            return carry

        lax.fori_loop(0, ZC, zrow, 0)
        for t in range(ZT):
            zc = s + t * NS

            @pl.when(jnp.logical_or(NZ % NS == 0, zc < NZ))
            def _():
                pltpu.async_copy(g_v.at[0], agg_sh.at[pl.ds(zc * ZC, ZC)],
                                 sem_s[0])
        for t in range(ZT):
            zc = s + t * NS

            @pl.when(jnp.logical_or(NZ % NS == 0, zc < NZ))
            def _():
                pltpu.make_async_copy(out0.at[pl.ds(0, ZC)], g_v.at[0],
                                      sem_s[0]).wait()
        plsc.subcore_barrier()

        for g0 in range(NBI - 1):
            issue_idx(g0, g0)
        for g0 in range(NB - 1):
            wait_idx(g0)
            issue_loads(g0, g0, g0)

        def step(t, carry):
            for u in range(NBI):
                g = t * NBI + u
                b = u % NB
                bi = u

                @pl.when(g < NCH)
                def _():
                    wait_loads(b)

                    def row(ri, rc):
                      for k in range(4):
                        r = ri * 4 + k
                        for j in range(DV // 2):
                            w = jax.lax.bitcast_convert_type(
                                ea_v[b, r, pl.ds(j * LANES, LANES)],
                                jnp.uint32)
                            e0 = jax.lax.bitcast_convert_type(
                                w << 16, jnp.float32)
                            e1 = jax.lax.bitcast_convert_type(
                                w & jnp.uint32(0xFFFF0000), jnp.float32)
                            sl0 = pl.ds(2 * j * LANES, LANES)
                            sl1 = pl.ds((2 * j + 1) * LANES, LANES)
                            g_v[b, r, sl0] = jnp.maximum(g_v[b, r, sl0] + e0,
                                                         0.0)
                            g_v[b, r, sl1] = jnp.maximum(g_v[b, r, sl1] + e1,
                                                         0.0)
                      return rc

                    lax.fori_loop(0, C // 4, row, 0)
                    issue_scatter(b, bi)

                    @pl.when(g + PF < NCH)
                    def _():
                        @pl.when(g >= 1)
                        def _():
                            wait_scatter((b + NB - 1) % NB)

                        @pl.when(g + NBI - 1 < NCH)
                        def _():
                            issue_idx(g + NBI - 1, (u + NBI - 1) % NBI)

                        wait_idx((u + PF) % NBI)
                        issue_loads(g + PF, (b + PF) % NB, (u + PF) % NBI)

            return carry

        lax.fori_loop(0, T, step, 0)
        for b in range(NB):
            wait_scatter(b)
        plsc.subcore_barrier()

        def co_read(t, p):
            zc = s + t * NS

            @pl.when(jnp.logical_or(NZ % NS == 0, zc < NZ))
            def _():
                pltpu.async_copy(agg_sh.at[pl.ds(zc * ZC, ZC)], g_v.at[p],
                                 sem_in[p])

        def co_wait_read(t, p):
            zc = s + t * NS

            @pl.when(jnp.logical_or(NZ % NS == 0, zc < NZ))
            def _():
                pltpu.make_async_copy(out0.at[pl.ds(0, ZC)], g_v.at[p],
                                      sem_in[p]).wait()

        def co_write(t, p):
            zc = s + t * NS

            @pl.when(jnp.logical_or(NZ % NS == 0, zc < NZ))
            def _():
                row0 = zc * ZC

                @pl.when(c == 0)
                def _():
                    pltpu.async_copy(g_v.at[p], out0.at[pl.ds(row0, ZC)],
                                     sem_s[p])

                @pl.when(c == 1)
                def _():
                    pltpu.async_copy(g_v.at[p], out1.at[pl.ds(row0, ZC)],
                                     sem_s[p])

        def co_wait_write(t, p):
            zc = s + t * NS

            @pl.when(jnp.logical_or(NZ % NS == 0, zc < NZ))
            def _():
                pltpu.make_async_copy(out0.at[pl.ds(0, ZC)], g_v.at[p],
                                      sem_s[p]).wait()

        co_read(0, 0)
        for t in range(ZT):
            p = t % 2
            if t + 1 < ZT:
                if t >= 1:
                    co_wait_write(t - 1, (t + 1) % 2)
                co_read(t + 1, (t + 1) % 2)
            co_wait_read(t, p)
            co_write(t, p)
        for t in (ZT - 2, ZT - 1):
            if t >= 0:
                co_wait_write(t, t % 2)

    return k


def kernel(x, edge_index, edge_attr, params):
    N, D = x.shape
    E = edge_index.shape[1]
    C = 40
    NCH = E // (NW * C)

    idx_r = edge_index.reshape(2, NW, NCH, C).transpose(1, 2, 0, 3)

    A = _a_perm(D)
    eye = np.eye(D, dtype=np.float32)
    e_a = jnp.asarray(eye[:, A])
    e_even = jnp.asarray(eye[:, np.arange(0, D, 2)])
    e_odd = jnp.asarray(eye[:, np.arange(1, D, 2)])

    nl = len(params)
    eas = [_ea_proj1(edge_attr, p['eW'], p['eb'], e_even, e_odd)
           for p in params]
    ha = _prep(x, e_a)
    sc = _sc_gather_scatter_fn(N, D, C, NCH)
    for l, p in enumerate(params):
        a0, a1 = sc(ha, idx_r, eas[l])
        if l < nl - 1:
            ha = _mlp_mid(ha, a0, a1, p, A)
        else:
            ha = _mlp_last(ha, a0, a1, p, A)
    return ha


# speedup vs baseline: 1.0267x; 1.0267x over previous
import functools

import jax
import jax.numpy as jnp
import numpy as np
from jax import lax
from jax.experimental import pallas as pl
from jax.experimental.pallas import tpu as pltpu
from jax.experimental.pallas import tpu_sc as plsc

NC = 2
NS = 16
NW = NC * NS
LANES = 16


def _a_perm(D):
    return np.concatenate(
        [np.concatenate([g * 32 + np.arange(0, 32, 2),
                         g * 32 + np.arange(1, 32, 2)])
         for g in range(D // 32)])


def _rne_bf16_bits(u):
    return (u + jnp.uint32(0x7FFF) + ((u >> 16) & jnp.uint32(1))) & jnp.uint32(
        0xFFFF0000)


def _pack_bf16(v, e_even, e_odd):
    pe = jnp.dot(v, e_even, preferred_element_type=jnp.float32)
    po = jnp.dot(v, e_odd, preferred_element_type=jnp.float32)
    ue = jax.lax.bitcast_convert_type(pe, jnp.uint32)
    uo = jax.lax.bitcast_convert_type(po, jnp.uint32)
    w = (_rne_bf16_bits(ue) >> 16) | _rne_bf16_bits(uo)
    return jax.lax.bitcast_convert_type(w, jnp.float32)


def _ea_proj_body(attr_ref, w_ref, b_ref, ee_ref, eo_ref, o0, o1, o2):
    for j, o in enumerate((o0, o1, o2)):
        ea = (jnp.dot(attr_ref[...], w_ref[..., j, :],
                      preferred_element_type=jnp.float32) + b_ref[..., j, :])
        o[...] = _pack_bf16(ea, ee_ref[...], eo_ref[...])


def _ea_proj3(edge_attr, ws, bs, e_even, e_odd, blk=2000):
    E, ED = edge_attr.shape
    D = ws[0].shape[1]
    half = pl.BlockSpec((blk, D // 2), lambda i: (i, 0))
    outs = pl.pallas_call(
        _ea_proj_body,
        grid=(E // blk,),
        in_specs=[
            pl.BlockSpec((blk, ED), lambda i: (i, 0)),
            pl.BlockSpec((ED, 3, D), lambda i: (0, 0, 0)),
            pl.BlockSpec((1, 3, D), lambda i: (0, 0, 0)),
            pl.BlockSpec((D, D // 2), lambda i: (0, 0)),
            pl.BlockSpec((D, D // 2), lambda i: (0, 0)),
        ],
        out_specs=[half, half, half],
        out_shape=[jax.ShapeDtypeStruct((E, D // 2), jnp.float32)] * 3,
    )(edge_attr, jnp.stack(ws, axis=1), jnp.stack(bs, axis=0).reshape(1, 3, D),
      e_even, e_odd)
    return outs


def _prep_body(x_ref, ea_ref, xa_ref):
    xa_ref[...] = jnp.dot(x_ref[...], ea_ref[...],
                          preferred_element_type=jnp.float32)


def _prep(x, e_a, blk=1000):
    N, D = x.shape
    mat = pl.BlockSpec((D, D), lambda i: (0, 0))
    rows = pl.BlockSpec((blk, D), lambda i: (i, 0))
    return pl.pallas_call(
        _prep_body,
        grid=(N // blk,),
        in_specs=[rows, mat],
        out_specs=rows,
        out_shape=jax.ShapeDtypeStruct((N, D), jnp.float32),
    )(x, e_a)


def _mlp_mid_body(x_ref, a0_ref, a1_ref, w0a, b0, w1, b1, w2a, b2a, ha_ref):
    h = x_ref[...] + a0_ref[...] + a1_ref[...]
    h = jnp.maximum(
        jnp.dot(h, w0a[...], preferred_element_type=jnp.float32) + b0[...], 0.0)
    h = jnp.maximum(
        jnp.dot(h, w1[...], preferred_element_type=jnp.float32) + b1[...], 0.0)
    h = jnp.dot(h, w2a[...], preferred_element_type=jnp.float32) + b2a[...]
    ha_ref[...] = jnp.maximum(h, 0.0)


def _mlp_mid(xa, a0, a1, p, A, blk=1000):
    N, D = xa.shape
    mat = pl.BlockSpec((D, D), lambda i: (0, 0))
    vec = pl.BlockSpec((1, D), lambda i: (0, 0))
    rows = pl.BlockSpec((blk, D), lambda i: (i, 0))
    return pl.pallas_call(
        _mlp_mid_body,
        grid=(N // blk,),
        in_specs=[rows, rows, rows, mat, vec, mat, vec, mat, vec],
        out_specs=rows,
        out_shape=jax.ShapeDtypeStruct((N, D), jnp.float32),
    )(xa, a0, a1,
      p['W0'][A, :], p['b0'].reshape(1, D),
      p['W1'], p['b1'].reshape(1, D),
      p['W2'][:, A], p['b2'][A].reshape(1, D))


def _mlp_last_body(x_ref, a0_ref, a1_ref, w0a, b0, w1, b1, w2, b2, out_ref):
    h = x_ref[...] + a0_ref[...] + a1_ref[...]
    h = jnp.maximum(
        jnp.dot(h, w0a[...], preferred_element_type=jnp.float32) + b0[...], 0.0)
    h = jnp.maximum(
        jnp.dot(h, w1[...], preferred_element_type=jnp.float32) + b1[...], 0.0)
    out_ref[...] = (
        jnp.dot(h, w2[...], preferred_element_type=jnp.float32) + b2[...])


def _mlp_last(xa, a0, a1, p, A, blk=1000):
    N, D = xa.shape
    mat = pl.BlockSpec((D, D), lambda i: (0, 0))
    vec = pl.BlockSpec((1, D), lambda i: (0, 0))
    rows = pl.BlockSpec((blk, D), lambda i: (i, 0))
    return pl.pallas_call(
        _mlp_last_body,
        grid=(N // blk,),
        in_specs=[rows, rows, rows, mat, vec, mat, vec, mat, vec],
        out_specs=rows,
        out_shape=jax.ShapeDtypeStruct((N, D), jnp.float32),
    )(xa, a0, a1,
      p['W0'][A, :], p['b0'].reshape(1, D),
      p['W1'], p['b1'].reshape(1, D),
      p['W2'], p['b2'].reshape(1, D))


@functools.lru_cache(maxsize=None)
def _sc_gather_scatter_fn(N, D, C, NCH):
    NB = 4
    NBI = 8
    PF = NB - 1
    ZC = C
    NZ = N // ZC
    ZT = (NZ + NS - 1) // NS
    DV = D // LANES
    T = (NCH + NBI - 1) // NBI

    mesh = plsc.VectorSubcoreMesh(core_axis_name="c", subcore_axis_name="s")

    @functools.partial(
        pl.kernel,
        out_type=[jax.ShapeDtypeStruct((N, D), jnp.float32),
                  jax.ShapeDtypeStruct((N, D), jnp.float32)],
        mesh=mesh,
        scratch_types=[
            pltpu.VMEM((NBI, 2, C), jnp.int32),
            pltpu.VMEM((NB, C, D // 2), jnp.float32),
            pltpu.VMEM((NB, C, D), jnp.float32),
            pltpu.VMEM_SHARED((N, D), jnp.float32),
            [pltpu.SemaphoreType.DMA] * NBI,
            [pltpu.SemaphoreType.DMA] * NB,
            [pltpu.SemaphoreType.DMA] * NB,
        ],
    )
    def k(x_hbm, idx_hbm, ea_hbm, out0, out1,
          idx_v, ea_v, g_v, agg_sh, sem_idx, sem_in, sem_s):
        c = lax.axis_index("c")
        s = lax.axis_index("s")
        wid = s * NC + c
        base = wid * NCH

        def issue_idx(g, bi):
            pltpu.async_copy(idx_hbm.at[wid, g], idx_v.at[bi], sem_idx[bi])

        def wait_idx(bi):
            pltpu.make_async_copy(idx_hbm.at[0, 0], idx_v.at[bi],
                                  sem_idx[bi]).wait()

        def issue_loads(g, b, bi):
            pltpu.async_copy(ea_hbm.at[pl.ds((base + g) * C, C)], ea_v.at[b],
                             sem_in[b])
            pltpu.async_copy(x_hbm.at[idx_v.at[bi, 0]], g_v.at[b], sem_in[b])

        def wait_loads(b):
            pltpu.make_async_copy(ea_hbm.at[pl.ds(0, C)], ea_v.at[b],
                                  sem_in[b]).wait()
            pltpu.make_async_copy(out0.at[pl.ds(0, C)], g_v.at[b],
                                  sem_in[b]).wait()

        def issue_scatter(b, bi):
            pltpu.async_copy(g_v.at[b], agg_sh.at[idx_v.at[bi, 1]], sem_s[b],
                             add=True)

        def wait_scatter(b):
            pltpu.make_async_copy(out0.at[pl.ds(0, C)], g_v.at[b],
                                  sem_s[b]).wait()

        zero = jnp.zeros((LANES,), jnp.float32)

        def zrow(r, carry):
            for j in range(DV):
                g_v[0, r, pl.ds(j * LANES, LANES)] = zero
            return carry

        lax.fori_loop(0, ZC, zrow, 0)
        for t in range(ZT):
            zc = s + t * NS

            @pl.when(jnp.logical_or(NZ % NS == 0, zc < NZ))
            def _():
                pltpu.async_copy(g_v.at[0], agg_sh.at[pl.ds(zc * ZC, ZC)],
                                 sem_s[0])
        for t in range(ZT):
            zc = s + t * NS

            @pl.when(jnp.logical_or(NZ % NS == 0, zc < NZ))
            def _():
                pltpu.make_async_copy(out0.at[pl.ds(0, ZC)], g_v.at[0],
                                      sem_s[0]).wait()
        plsc.subcore_barrier()

        for g0 in range(NBI - 1):
            issue_idx(g0, g0)
        for g0 in range(NB - 1):
            wait_idx(g0)
            issue_loads(g0, g0, g0)

        def step(t, carry):
            for u in range(NBI):
                g = t * NBI + u
                b = u % NB
                bi = u

                @pl.when(g < NCH)
                def _():
                    wait_loads(b)

                    def row(ri, rc):
                      for k in range(4):
                        r = ri * 4 + k
                        for j in range(DV // 2):
                            w = jax.lax.bitcast_convert_type(
                                ea_v[b, r, pl.ds(j * LANES, LANES)],
                                jnp.uint32)
                            e0 = jax.lax.bitcast_convert_type(
                                w << 16, jnp.float32)
                            e1 = jax.lax.bitcast_convert_type(
                                w & jnp.uint32(0xFFFF0000), jnp.float32)
                            sl0 = pl.ds(2 * j * LANES, LANES)
                            sl1 = pl.ds((2 * j + 1) * LANES, LANES)
                            g_v[b, r, sl0] = jnp.maximum(g_v[b, r, sl0] + e0,
                                                         0.0)
                            g_v[b, r, sl1] = jnp.maximum(g_v[b, r, sl1] + e1,
                                                         0.0)
                      return rc

                    lax.fori_loop(0, C // 4, row, 0)
                    issue_scatter(b, bi)

                    @pl.when(g + PF < NCH)
                    def _():
                        @pl.when(g >= 1)
                        def _():
                            wait_scatter((b + NB - 1) % NB)

                        @pl.when(g + NBI - 1 < NCH)
                        def _():
                            issue_idx(g + NBI - 1, (u + NBI - 1) % NBI)

                        wait_idx((u + PF) % NBI)
                        issue_loads(g + PF, (b + PF) % NB, (u + PF) % NBI)

            return carry

        lax.fori_loop(0, T, step, 0)
        for b in range(NB):
            wait_scatter(b)
        plsc.subcore_barrier()

        def co_read(t, p):
            zc = s + t * NS

            @pl.when(jnp.logical_or(NZ % NS == 0, zc < NZ))
            def _():
                pltpu.async_copy(agg_sh.at[pl.ds(zc * ZC, ZC)], g_v.at[p],
                                 sem_in[p])

        def co_wait_read(t, p):
            zc = s + t * NS

            @pl.when(jnp.logical_or(NZ % NS == 0, zc < NZ))
            def _():
                pltpu.make_async_copy(out0.at[pl.ds(0, ZC)], g_v.at[p],
                                      sem_in[p]).wait()

        def co_write(t, p):
            zc = s + t * NS

            @pl.when(jnp.logical_or(NZ % NS == 0, zc < NZ))
            def _():
                row0 = zc * ZC

                @pl.when(c == 0)
                def _():
                    pltpu.async_copy(g_v.at[p], out0.at[pl.ds(row0, ZC)],
                                     sem_s[p])

                @pl.when(c == 1)
                def _():
                    pltpu.async_copy(g_v.at[p], out1.at[pl.ds(row0, ZC)],
                                     sem_s[p])

        def co_wait_write(t, p):
            zc = s + t * NS

            @pl.when(jnp.logical_or(NZ % NS == 0, zc < NZ))
            def _():
                pltpu.make_async_copy(out0.at[pl.ds(0, ZC)], g_v.at[p],
                                      sem_s[p]).wait()

        co_read(0, 0)
        for t in range(ZT):
            p = t % 2
            if t + 1 < ZT:
                if t >= 1:
                    co_wait_write(t - 1, (t + 1) % 2)
                co_read(t + 1, (t + 1) % 2)
            co_wait_read(t, p)
            co_write(t, p)
        for t in (ZT - 2, ZT - 1):
            if t >= 0:
                co_wait_write(t, t % 2)

    return k


def kernel(x, edge_index, edge_attr, params):
    N, D = x.shape
    E = edge_index.shape[1]
    C = 40
    NCH = E // (NW * C)

    idx_r = edge_index.reshape(2, NW, NCH, C).transpose(1, 2, 0, 3)

    A = _a_perm(D)
    eye = np.eye(D, dtype=np.float32)
    e_a = jnp.asarray(eye[:, A])
    e_even = jnp.asarray(eye[:, np.arange(0, D, 2)])
    e_odd = jnp.asarray(eye[:, np.arange(1, D, 2)])

    nl = len(params)
    eas = _ea_proj3(edge_attr, [p['eW'] for p in params],
                    [p['eb'] for p in params], e_even, e_odd)
    ha = _prep(x, e_a)
    sc = _sc_gather_scatter_fn(N, D, C, NCH)
    for l, p in enumerate(params):
        a0, a1 = sc(ha, idx_r, eas[l])
        if l < nl - 1:
            ha = _mlp_mid(ha, a0, a1, p, A)
        else:
            ha = _mlp_last(ha, a0, a1, p, A)
    return ha
